# baseline (speedup 1.0000x reference)
import functools

import jax
import jax.numpy as jnp
from jax import lax
from jax.experimental import pallas as pl
from jax.experimental.pallas import tpu as pltpu
from jax.experimental.pallas import tpu_sc as plsc

KNN = 20
ROW_TILE = 256
NEG = -3.0e38
BIG = 1 << 30
BN_EPS_K = 1e-5
NW = 32
CHK = 16
CCH = 24
CAND = CHK * CCH
SC_CHUNK = 128


def _dist_body(x_ref, xf_ref, wi_ref, d_ref, gcid_ref, ui_ref):
    x = x_ref[0]
    xf = xf_ref[0]
    r = x.shape[0]
    n = xf.shape[0]
    nchk = n // CHK
    contract = (((1,), (1,)), ((), ()))
    xb = x.astype(jnp.bfloat16)
    gram = lax.dot_general(xb, xf.astype(jnp.bfloat16), contract,
                           preferred_element_type=jnp.float32)
    xin = -2.0 * gram
    rowsq = jnp.sum(x * x, axis=1, keepdims=True)
    colsq = jnp.sum(xf * xf, axis=1)[None, :]
    d = (-rowsq - xin) - colsq
    d_ref[0] = d
    ui_ref[0] = lax.dot_general(xb, wi_ref[...], contract,
                                preferred_element_type=jnp.float32)

    cmax = jnp.max(d.reshape(r, nchk, CHK), axis=2)
    ciota = lax.broadcasted_iota(jnp.int32, (r, nchk), 1)
    c24 = lax.broadcasted_iota(jnp.int32, (r, CCH), 1)

    def cstep(t, carry):
        m_val, acc = carry
        m = jnp.max(m_val, axis=1, keepdims=True)
        eqc = m_val == m
        cid = jnp.min(jnp.where(eqc, ciota, nchk), axis=1, keepdims=True)
        m_val = jnp.where(ciota == cid, NEG, m_val)
        acc = jnp.where(c24 == t, cid, acc)
        return m_val, acc

    _, cids = lax.fori_loop(0, CCH, cstep,
                            (cmax, jnp.zeros((r, CCH), jnp.int32)))
    row0 = pl.program_id(0) * n + pl.program_id(1) * r
    riota = lax.broadcasted_iota(jnp.int32, (r, CCH), 0) + row0
    gcid_ref[0] = riota * nchk + cids


def _tc_dist(xT, wi_bf):
    b, n, cin = xT.shape
    g = wi_bf.shape[0]
    r = ROW_TILE
    return pl.pallas_call(
        _dist_body,
        grid=(b, n // r),
        in_specs=[
            pl.BlockSpec((1, r, cin), lambda i, j: (i, j, 0)),
            pl.BlockSpec((1, n, cin), lambda i, j: (i, 0, 0)),
            pl.BlockSpec((g, cin), lambda i, j: (0, 0)),
        ],
        out_specs=[
            pl.BlockSpec((1, r, n), lambda i, j: (i, j, 0)),
            pl.BlockSpec((1, r, CCH), lambda i, j: (i, j, 0)),
            pl.BlockSpec((1, r, g), lambda i, j: (i, j, 0)),
        ],
        out_shape=[
            jax.ShapeDtypeStruct((b, n, n), jnp.float32),
            jax.ShapeDtypeStruct((b, n, CCH), jnp.int32),
            jax.ShapeDtypeStruct((b, n, g), jnp.float32),
        ],
    )(xT, xT, wi_bf)


def _pick_body(dc_ref, gcid_ref, idx_ref, *, n):
    dc = dc_ref[0]
    gc = gcid_ref[0]
    r = dc.shape[0]
    nchk = n // CHK
    boff = pl.program_id(0) * n
    cid = jnp.bitwise_and(gc, nchk - 1)
    imap = ((cid * CHK)[:, :, None]
            + lax.broadcasted_iota(jnp.int32, (r, CCH, CHK), 2))
    imap = imap.reshape(r, CAND) + boff
    kiota = lax.broadcasted_iota(jnp.int32, (r, KNN), 1)

    def step(t, carry):
        d, acc = carry
        m = jnp.max(d, axis=1, keepdims=True)
        eq = d == m
        jst = jnp.min(jnp.where(eq, imap, BIG), axis=1, keepdims=True)
        d = jnp.where(imap == jst, NEG, d)
        acc = jnp.where(kiota == t, jst, acc)
        return d, acc

    _, idx = lax.fori_loop(0, KNN, step,
                           (dc, jnp.zeros((r, KNN), jnp.int32)))
    idx_ref[0] = idx


def _tc_pick(dcomp, gcid, b, n):
    cand = dcomp.shape[1]
    r = ROW_TILE
    return pl.pallas_call(
        functools.partial(_pick_body, n=n),
        grid=(b, n // r),
        in_specs=[
            pl.BlockSpec((1, r, cand), lambda i, j: (i, j, 0)),
            pl.BlockSpec((1, r, CCH), lambda i, j: (i, j, 0)),
        ],
        out_specs=pl.BlockSpec((1, r, KNN), lambda i, j: (i, j, 0)),
        out_shape=jax.ShapeDtypeStruct((b, n, KNN), jnp.int32),
    )(dcomp.reshape(b, n, cand), gcid.reshape(b, n, CCH))


def _sc_gather(table, ids):
    t_rows, w = table.shape
    l = ids.shape[0]
    lpw = l // NW
    ch = lpw // SC_CHUNK
    idx_r = ids.reshape(NW, ch, SC_CHUNK)

    @functools.partial(
        pl.kernel,
        out_type=jax.ShapeDtypeStruct((NW, ch, SC_CHUNK, w), jnp.float32),
        mesh=plsc.VectorSubcoreMesh(core_axis_name="c", subcore_axis_name="s"),
        compiler_params=pltpu.CompilerParams(use_tc_tiling_on_sc=False),
        scratch_types=[
            pltpu.VMEM((ch, SC_CHUNK), jnp.int32),
            pltpu.VMEM((2, SC_CHUNK, w), jnp.float32),
            pltpu.SemaphoreType.DMA,
            pltpu.SemaphoreType.DMA,
            pltpu.SemaphoreType.DMA,
            pltpu.SemaphoreType.DMA,
        ],
    )
    def sc_kern(table_hbm, idx_hbm, out_hbm, idx_v, rows_v, gs0, gs1, os0, os1):
        wid = lax.axis_index("s") * 2 + lax.axis_index("c")
        pltpu.sync_copy(idx_hbm.at[wid], idx_v)
        gsems = (gs0, gs1)
        osems = (os0, os1)

        def g_start(c, buf):
            pltpu.async_copy(table_hbm.at[idx_v.at[c]], rows_v.at[buf], gsems[buf])

        def g_wait(c, buf):
            pltpu.make_async_copy(table_hbm.at[idx_v.at[c]], rows_v.at[buf],
                                  gsems[buf]).wait()

        def o_start(c, buf):
            pltpu.async_copy(rows_v.at[buf], out_hbm.at[wid, c], osems[buf])

        def o_wait(c, buf):
            pltpu.make_async_copy(rows_v.at[buf], out_hbm.at[wid, c],
                                  osems[buf]).wait()

        g_start(0, 0)
        g_start(1, 1)

        def loop_body(c2, carry):
            c = 2 * c2
            for buf in (0, 1):
                cc = c + buf
                g_wait(cc, buf)
                o_start(cc, buf)

                @pl.when(cc + 2 < ch)
                def _():
                    o_wait(cc, buf)
                    g_start(cc + 2, buf)

            return carry

        lax.fori_loop(0, ch // 2, loop_body, jnp.int32(0))
        o_wait(ch - 2, 0)
        o_wait(ch - 1, 1)

    return sc_kern(table, idx_r).reshape(l, w)


def _edge_body(xj_ref, x_ref, wd_ref, ui_ref, bn_ref, f_ref):
    r = x_ref.shape[1]
    xj = xj_ref[0]
    xi = x_ref[0]
    cin = xi.shape[1]
    g = ui_ref.shape[2]
    xi_b = jnp.broadcast_to(xi[:, None, :], (r, KNN, cin)).reshape(r * KNN, cin)
    e = (xj - xi_b).astype(jnp.bfloat16)
    ed = lax.dot_general(e, wd_ref[...], (((1,), (1,)), ((), ())),
                         preferred_element_type=jnp.float32)
    bias = bn_ref[0][None, None, :]
    gamma = bn_ref[1][None, None, :]
    beta = bn_ref[2][None, None, :]
    y = ed.reshape(r, KNN, g) + ui_ref[0][:, None, :]
    y = y + bias
    y = y / jnp.sqrt(jnp.float32(1.0) + jnp.float32(BN_EPS_K))
    y = y * gamma + beta
    y = jnp.maximum(y, 0.0)
    f_ref[0] = jnp.max(y, axis=1)


def _tc_edge(xj, xT, wd_bf, ui, bn):
    b, n, cin = xT.shape
    g = wd_bf.shape[0]
    r = ROW_TILE
    return pl.pallas_call(
        _edge_body,
        grid=(b, n // r),
        in_specs=[
            pl.BlockSpec((1, r * KNN, cin), lambda i, j: (i, j, 0)),
            pl.BlockSpec((1, r, cin), lambda i, j: (i, j, 0)),
            pl.BlockSpec((g, cin), lambda i, j: (0, 0)),
            pl.BlockSpec((1, r, g), lambda i, j: (i, j, 0)),
            pl.BlockSpec((3, g), lambda i, j: (0, 0)),
        ],
        out_specs=pl.BlockSpec((1, r, g), lambda i, j: (i, j, 0)),
        out_shape=jax.ShapeDtypeStruct((b, n, g), jnp.float32),
    )(xj.reshape(b, n * KNN, cin), xT, wd_bf, ui, bn)


def _edge_block(xT, w, bias, gamma, beta):
    b, n, cin = xT.shape
    g = w.shape[0]
    m = b * n
    wi_bf = w[:, :cin].astype(jnp.bfloat16)
    wd_bf = w[:, cin:].astype(jnp.bfloat16)
    bn = jnp.stack([bias, gamma, beta])
    d, gcid, ui = _tc_dist(xT, wi_bf)
    dcomp = _sc_gather(d.reshape(m * (n // CHK), CHK), gcid.reshape(m * CCH))
    idx = _tc_pick(dcomp.reshape(m, CAND), gcid.reshape(m, CCH), b, n)
    xj = _sc_gather(xT.reshape(m, cin), idx.reshape(m * KNN))
    return _tc_edge(xj, xT, wd_bf, ui, bn)


def kernel(inputs, W0, b0, gamma0, beta0, W1, b1, gamma1, beta1, W2, b2, gamma2, beta2):
    x0 = jnp.transpose(inputs[..., 0], (0, 2, 1))
    f0 = _edge_block(x0, W0, b0, gamma0, beta0)
    x1 = jnp.concatenate([f0, x0], axis=-1)
    f1 = _edge_block(x1, W1, b1, gamma1, beta1)
    x2 = jnp.concatenate([f1, x1], axis=-1)
    f2 = _edge_block(x2, W2, b2, gamma2, beta2)
    out = jnp.concatenate([f0, f1, f2, x0], axis=-1)
    return jnp.transpose(out, (0, 2, 1))[..., None]


# speedup vs baseline: 1.5986x; 1.5986x over previous
import functools

import jax
import jax.numpy as jnp
from jax import lax
from jax.experimental import pallas as pl
from jax.experimental.pallas import tpu as pltpu
from jax.experimental.pallas import tpu_sc as plsc

KNN = 20
ROW_TILE = 256
NEG = -3.0e38
BN_EPS_K = 1e-5
NW = 32
CPD = 4
CI = CPD * KNN


def _topk_body(x_ref, xf_ref, wi_ref, idx_ref, ui_ref, d_scr):
    x = x_ref[0]
    xf = xf_ref[0]
    r = x.shape[0]
    n = xf.shape[0]
    contract = (((1,), (1,)), ((), ()))
    xb = x.astype(jnp.bfloat16)
    gram = lax.dot_general(xb, xf.astype(jnp.bfloat16), contract,
                           preferred_element_type=jnp.float32)
    xin = -2.0 * gram
    rowsq = jnp.sum(x * x, axis=1, keepdims=True)
    colsq = jnp.sum(xf * xf, axis=1)[None, :]
    d_scr[...] = (-rowsq - xin) - colsq
    ui_ref[0] = lax.dot_general(xb, wi_ref[...], contract,
                                preferred_element_type=jnp.float32)

    boff = pl.program_id(0) * n
    iota = lax.broadcasted_iota(jnp.int32, (r, n), 1)
    kiota = lax.broadcasted_iota(jnp.int32, (r, KNN), 1)

    def step(t, carry):
        m_prev, acc = carry
        d = d_scr[...]
        m = jnp.max(jnp.where(d < m_prev, d, NEG), axis=1, keepdims=True)
        jstar = jnp.min(jnp.where(d == m, iota, n), axis=1, keepdims=True)
        acc = jnp.where(kiota == t, jstar + boff, acc)
        return m, acc

    _, idx = lax.fori_loop(
        0, KNN, step,
        (jnp.full((r, 1), jnp.inf, jnp.float32),
         jnp.full((r, KNN), boff, jnp.int32)))
    idx_ref[0] = idx


def _tc_topk(xT, wi_bf):
    b, n, cin = xT.shape
    g = wi_bf.shape[0]
    r = ROW_TILE
    return pl.pallas_call(
        _topk_body,
        grid=(b, n // r),
        in_specs=[
            pl.BlockSpec((1, r, cin), lambda i, j: (i, j, 0)),
            pl.BlockSpec((1, n, cin), lambda i, j: (i, 0, 0)),
            pl.BlockSpec((g, cin), lambda i, j: (0, 0)),
        ],
        out_specs=[
            pl.BlockSpec((1, r, KNN), lambda i, j: (i, j, 0)),
            pl.BlockSpec((1, r, g), lambda i, j: (i, j, 0)),
        ],
        out_shape=[
            jax.ShapeDtypeStruct((b, n, KNN), jnp.int32),
            jax.ShapeDtypeStruct((b, n, g), jnp.float32),
        ],
        scratch_shapes=[pltpu.VMEM((r, n), jnp.float32)],
    )(xT, xT, wi_bf)


def _sc_gather(table, idx):
    m, cin = table.shape
    ppw = m // NW
    ch = ppw // CPD
    idx_r = idx.reshape(NW, ch, CI)

    @functools.partial(
        pl.kernel,
        out_type=jax.ShapeDtypeStruct((NW, ch, CI, cin), jnp.float32),
        mesh=plsc.VectorSubcoreMesh(core_axis_name="c", subcore_axis_name="s"),
        compiler_params=pltpu.CompilerParams(use_tc_tiling_on_sc=False),
        scratch_types=[
            pltpu.VMEM((ch, CI), jnp.int32),
            pltpu.VMEM((2, CI, cin), jnp.float32),
            pltpu.SemaphoreType.DMA,
            pltpu.SemaphoreType.DMA,
            pltpu.SemaphoreType.DMA,
            pltpu.SemaphoreType.DMA,
        ],
    )
    def sc_kern(table_hbm, idx_hbm, out_hbm, idx_v, rows_v, gs0, gs1, os0, os1):
        wid = lax.axis_index("s") * 2 + lax.axis_index("c")
        pltpu.sync_copy(idx_hbm.at[wid], idx_v)
        gsems = (gs0, gs1)
        osems = (os0, os1)

        def g_start(c, buf):
            pltpu.async_copy(table_hbm.at[idx_v.at[c]], rows_v.at[buf], gsems[buf])

        def g_wait(c, buf):
            pltpu.make_async_copy(table_hbm.at[idx_v.at[c]], rows_v.at[buf],
                                  gsems[buf]).wait()

        def o_start(c, buf):
            pltpu.async_copy(rows_v.at[buf], out_hbm.at[wid, c], osems[buf])

        def o_wait(c, buf):
            pltpu.make_async_copy(rows_v.at[buf], out_hbm.at[wid, c],
                                  osems[buf]).wait()

        g_start(0, 0)
        g_start(1, 1)

        def loop_body(c2, carry):
            c = 2 * c2
            for buf in (0, 1):
                cc = c + buf
                g_wait(cc, buf)
                o_start(cc, buf)

                @pl.when(cc + 2 < ch)
                def _():
                    o_wait(cc, buf)
                    g_start(cc + 2, buf)

            return carry

        lax.fori_loop(0, ch // 2, loop_body, jnp.int32(0))
        o_wait(ch - 2, 0)
        o_wait(ch - 1, 1)

    return sc_kern(table, idx_r).reshape(m * KNN, cin)


def _edge_body(xj_ref, x_ref, wd_ref, ui_ref, bn_ref, f_ref):
    r = x_ref.shape[1]
    xj = xj_ref[0]
    xi = x_ref[0]
    cin = xi.shape[1]
    g = ui_ref.shape[2]
    xi_b = jnp.broadcast_to(xi[:, None, :], (r, KNN, cin)).reshape(r * KNN, cin)
    e = (xj - xi_b).astype(jnp.bfloat16)
    ed = lax.dot_general(e, wd_ref[...], (((1,), (1,)), ((), ())),
                         preferred_element_type=jnp.float32)
    bias = bn_ref[0][None, None, :]
    gamma = bn_ref[1][None, None, :]
    beta = bn_ref[2][None, None, :]
    y = ed.reshape(r, KNN, g) + ui_ref[0][:, None, :]
    y = y + bias
    y = y / jnp.sqrt(jnp.float32(1.0) + jnp.float32(BN_EPS_K))
    y = y * gamma + beta
    y = jnp.maximum(y, 0.0)
    f_ref[0] = jnp.max(y, axis=1)


def _tc_edge(xj, xT, wd_bf, ui, bn):
    b, n, cin = xT.shape
    g = wd_bf.shape[0]
    r = ROW_TILE
    return pl.pallas_call(
        _edge_body,
        grid=(b, n // r),
        in_specs=[
            pl.BlockSpec((1, r * KNN, cin), lambda i, j: (i, j, 0)),
            pl.BlockSpec((1, r, cin), lambda i, j: (i, j, 0)),
            pl.BlockSpec((g, cin), lambda i, j: (0, 0)),
            pl.BlockSpec((1, r, g), lambda i, j: (i, j, 0)),
            pl.BlockSpec((3, g), lambda i, j: (0, 0)),
        ],
        out_specs=pl.BlockSpec((1, r, g), lambda i, j: (i, j, 0)),
        out_shape=jax.ShapeDtypeStruct((b, n, g), jnp.float32),
    )(xj.reshape(b, n * KNN, cin), xT, wd_bf, ui, bn)


def _edge_block(xT, w, bias, gamma, beta):
    b, n, cin = xT.shape
    g = w.shape[0]
    wi_bf = w[:, :cin].astype(jnp.bfloat16)
    wd_bf = w[:, cin:].astype(jnp.bfloat16)
    bn = jnp.stack([bias, gamma, beta])
    idx, ui = _tc_topk(xT, wi_bf)
    xj = _sc_gather(xT.reshape(b * n, cin), idx.reshape(b * n, KNN))
    return _tc_edge(xj, xT, wd_bf, ui, bn)


def kernel(inputs, W0, b0, gamma0, beta0, W1, b1, gamma1, beta1, W2, b2, gamma2, beta2):
    x0 = jnp.transpose(inputs[..., 0], (0, 2, 1))
    f0 = _edge_block(x0, W0, b0, gamma0, beta0)
    x1 = jnp.concatenate([f0, x0], axis=-1)
    f1 = _edge_block(x1, W1, b1, gamma1, beta1)
    x2 = jnp.concatenate([f1, x1], axis=-1)
    f2 = _edge_block(x2, W2, b2, gamma2, beta2)
    out = jnp.concatenate([f0, f1, f2, x0], axis=-1)
    return jnp.transpose(out, (0, 2, 1))[..., None]


# speedup vs baseline: 1.8589x; 1.1628x over previous
import functools

import jax
import jax.numpy as jnp
from jax import lax
from jax.experimental import pallas as pl
from jax.experimental.pallas import tpu as pltpu
from jax.experimental.pallas import tpu_sc as plsc

KNN = 20
ROW_TILE = 512
NEG = -3.0e38
BN_EPS_K = 1e-5
NW = 32
CPD = 4
CI = CPD * KNN


def _topk_body(x_ref, xf_ref, wi_ref, idx_ref, ui_ref, d_scr):
    x = x_ref[0]
    xf = xf_ref[0]
    r = x.shape[0]
    n = xf.shape[0]
    contract = (((1,), (1,)), ((), ()))
    xb = x.astype(jnp.bfloat16)
    gram = lax.dot_general(xb, xf.astype(jnp.bfloat16), contract,
                           preferred_element_type=jnp.float32)
    xin = -2.0 * gram
    rowsq = jnp.sum(x * x, axis=1, keepdims=True)
    colsq = jnp.sum(xf * xf, axis=1)[None, :]
    d_scr[...] = (-rowsq - xin) - colsq
    ui_ref[0] = lax.dot_general(xb, wi_ref[...], contract,
                                preferred_element_type=jnp.float32)

    boff = pl.program_id(0) * n
    iota = lax.broadcasted_iota(jnp.int32, (r, n), 1)
    kiota = lax.broadcasted_iota(jnp.int32, (r, KNN), 1)

    def step(t, acc):
        d = d_scr[...]
        m = jnp.max(d, axis=1, keepdims=True)
        eq = d == m
        jstar = jnp.min(jnp.where(eq, iota, n), axis=1, keepdims=True)
        d_scr[...] = jnp.where(eq, NEG, d)
        return jnp.where(kiota == t, jstar + boff, acc)

    idx_ref[0] = lax.fori_loop(0, KNN, step, jnp.full((r, KNN), boff, jnp.int32))


def _tc_topk(xT, wi_bf):
    b, n, cin = xT.shape
    g = wi_bf.shape[0]
    r = ROW_TILE
    return pl.pallas_call(
        _topk_body,
        grid=(b, n // r),
        in_specs=[
            pl.BlockSpec((1, r, cin), lambda i, j: (i, j, 0)),
            pl.BlockSpec((1, n, cin), lambda i, j: (i, 0, 0)),
            pl.BlockSpec((g, cin), lambda i, j: (0, 0)),
        ],
        out_specs=[
            pl.BlockSpec((1, r, KNN), lambda i, j: (i, j, 0)),
            pl.BlockSpec((1, r, g), lambda i, j: (i, j, 0)),
        ],
        out_shape=[
            jax.ShapeDtypeStruct((b, n, KNN), jnp.int32),
            jax.ShapeDtypeStruct((b, n, g), jnp.float32),
        ],
        scratch_shapes=[pltpu.VMEM((r, n), jnp.float32)],
    )(xT, xT, wi_bf)


def _sc_gather(table, idx):
    m, cin = table.shape
    ppw = m // NW
    ch = ppw // CPD
    idx_r = idx.reshape(NW, ch, CI)

    @functools.partial(
        pl.kernel,
        out_type=jax.ShapeDtypeStruct((NW, ch, CI, cin), jnp.float32),
        mesh=plsc.VectorSubcoreMesh(core_axis_name="c", subcore_axis_name="s"),
        compiler_params=pltpu.CompilerParams(use_tc_tiling_on_sc=False),
        scratch_types=[
            pltpu.VMEM((ch, CI), jnp.int32),
            pltpu.VMEM((2, CI, cin), jnp.float32),
            pltpu.SemaphoreType.DMA,
            pltpu.SemaphoreType.DMA,
            pltpu.SemaphoreType.DMA,
            pltpu.SemaphoreType.DMA,
        ],
    )
    def sc_kern(table_hbm, idx_hbm, out_hbm, idx_v, rows_v, gs0, gs1, os0, os1):
        wid = lax.axis_index("s") * 2 + lax.axis_index("c")
        pltpu.sync_copy(idx_hbm.at[wid], idx_v)
        gsems = (gs0, gs1)
        osems = (os0, os1)

        def g_start(c, buf):
            pltpu.async_copy(table_hbm.at[idx_v.at[c]], rows_v.at[buf], gsems[buf])

        def g_wait(c, buf):
            pltpu.make_async_copy(table_hbm.at[idx_v.at[c]], rows_v.at[buf],
                                  gsems[buf]).wait()

        def o_start(c, buf):
            pltpu.async_copy(rows_v.at[buf], out_hbm.at[wid, c], osems[buf])

        def o_wait(c, buf):
            pltpu.make_async_copy(rows_v.at[buf], out_hbm.at[wid, c],
                                  osems[buf]).wait()

        g_start(0, 0)
        g_start(1, 1)

        def loop_body(c2, carry):
            c = 2 * c2
            for buf in (0, 1):
                cc = c + buf
                g_wait(cc, buf)
                o_start(cc, buf)

                @pl.when(cc + 2 < ch)
                def _():
                    o_wait(cc, buf)
                    g_start(cc + 2, buf)

            return carry

        lax.fori_loop(0, ch // 2, loop_body, jnp.int32(0))
        o_wait(ch - 2, 0)
        o_wait(ch - 1, 1)

    return sc_kern(table, idx_r).reshape(m * KNN, cin)


def _edge_body(xj_ref, x_ref, wd_ref, ui_ref, bn_ref, f_ref):
    r = x_ref.shape[1]
    xj = xj_ref[0]
    xi = x_ref[0]
    cin = xi.shape[1]
    g = ui_ref.shape[2]
    xi_b = jnp.broadcast_to(xi[:, None, :], (r, KNN, cin)).reshape(r * KNN, cin)
    e = (xj - xi_b).astype(jnp.bfloat16)
    ed = lax.dot_general(e, wd_ref[...], (((1,), (1,)), ((), ())),
                         preferred_element_type=jnp.float32)
    bias = bn_ref[0][None, None, :]
    gamma = bn_ref[1][None, None, :]
    beta = bn_ref[2][None, None, :]
    y = ed.reshape(r, KNN, g) + ui_ref[0][:, None, :]
    y = y + bias
    y = y / jnp.sqrt(jnp.float32(1.0) + jnp.float32(BN_EPS_K))
    y = y * gamma + beta
    y = jnp.maximum(y, 0.0)
    f_ref[0] = jnp.max(y, axis=1)


def _tc_edge(xj, xT, wd_bf, ui, bn):
    b, n, cin = xT.shape
    g = wd_bf.shape[0]
    r = ROW_TILE
    return pl.pallas_call(
        _edge_body,
        grid=(b, n // r),
        in_specs=[
            pl.BlockSpec((1, r * KNN, cin), lambda i, j: (i, j, 0)),
            pl.BlockSpec((1, r, cin), lambda i, j: (i, j, 0)),
            pl.BlockSpec((g, cin), lambda i, j: (0, 0)),
            pl.BlockSpec((1, r, g), lambda i, j: (i, j, 0)),
            pl.BlockSpec((3, g), lambda i, j: (0, 0)),
        ],
        out_specs=pl.BlockSpec((1, r, g), lambda i, j: (i, j, 0)),
        out_shape=jax.ShapeDtypeStruct((b, n, g), jnp.float32),
    )(xj.reshape(b, n * KNN, cin), xT, wd_bf, ui, bn)


def _edge_block(xT, w, bias, gamma, beta):
    b, n, cin = xT.shape
    g = w.shape[0]
    wi_bf = w[:, :cin].astype(jnp.bfloat16)
    wd_bf = w[:, cin:].astype(jnp.bfloat16)
    bn = jnp.stack([bias, gamma, beta])
    idx, ui = _tc_topk(xT, wi_bf)
    xj = _sc_gather(xT.reshape(b * n, cin), idx.reshape(b * n, KNN))
    return _tc_edge(xj, xT, wd_bf, ui, bn)


def kernel(inputs, W0, b0, gamma0, beta0, W1, b1, gamma1, beta1, W2, b2, gamma2, beta2):
    x0 = jnp.transpose(inputs[..., 0], (0, 2, 1))
    f0 = _edge_block(x0, W0, b0, gamma0, beta0)
    x1 = jnp.concatenate([f0, x0], axis=-1)
    f1 = _edge_block(x1, W1, b1, gamma1, beta1)
    x2 = jnp.concatenate([f1, x1], axis=-1)
    f2 = _edge_block(x2, W2, b2, gamma2, beta2)
    out = jnp.concatenate([f0, f1, f2, x0], axis=-1)
    return jnp.transpose(out, (0, 2, 1))[..., None]


# speedup vs baseline: 1.8738x; 1.0080x over previous
import functools

import jax
import jax.numpy as jnp
from jax import lax
from jax.experimental import pallas as pl
from jax.experimental.pallas import tpu as pltpu
from jax.experimental.pallas import tpu_sc as plsc

KNN = 20
ROW_TILE = 1024
NEG = -3.0e38
BN_EPS_K = 1e-5
NW = 32
CPD = 4
CI = CPD * KNN


def _topk_body(x_ref, xf_ref, wi_ref, idx_ref, ui_ref, d_scr):
    x = x_ref[0]
    xf = xf_ref[0]
    r = x.shape[0]
    n = xf.shape[0]
    contract = (((1,), (1,)), ((), ()))
    xb = x.astype(jnp.bfloat16)
    gram = lax.dot_general(xb, xf.astype(jnp.bfloat16), contract,
                           preferred_element_type=jnp.float32)
    xin = -2.0 * gram
    rowsq = jnp.sum(x * x, axis=1, keepdims=True)
    colsq = jnp.sum(xf * xf, axis=1)[None, :]
    d_scr[...] = (-rowsq - xin) - colsq
    ui_ref[0] = lax.dot_general(xb, wi_ref[...], contract,
                                preferred_element_type=jnp.float32)

    boff = pl.program_id(0) * n
    iota = lax.broadcasted_iota(jnp.int32, (r, n), 1)
    kiota = lax.broadcasted_iota(jnp.int32, (r, KNN), 1)

    def step(t, acc):
        d = d_scr[...]
        m = jnp.max(d, axis=1, keepdims=True)
        eq = d == m
        jstar = jnp.min(jnp.where(eq, iota, n), axis=1, keepdims=True)
        d_scr[...] = jnp.where(eq, NEG, d)
        return jnp.where(kiota == t, jstar + boff, acc)

    idx_ref[0] = lax.fori_loop(0, KNN, step, jnp.full((r, KNN), boff, jnp.int32))


def _tc_topk(xT, wi_bf):
    b, n, cin = xT.shape
    g = wi_bf.shape[0]
    r = ROW_TILE
    return pl.pallas_call(
        _topk_body,
        grid=(b, n // r),
        in_specs=[
            pl.BlockSpec((1, r, cin), lambda i, j: (i, j, 0)),
            pl.BlockSpec((1, n, cin), lambda i, j: (i, 0, 0)),
            pl.BlockSpec((g, cin), lambda i, j: (0, 0)),
        ],
        out_specs=[
            pl.BlockSpec((1, r, KNN), lambda i, j: (i, j, 0)),
            pl.BlockSpec((1, r, g), lambda i, j: (i, j, 0)),
        ],
        out_shape=[
            jax.ShapeDtypeStruct((b, n, KNN), jnp.int32),
            jax.ShapeDtypeStruct((b, n, g), jnp.float32),
        ],
        scratch_shapes=[pltpu.VMEM((r, n), jnp.float32)],
    )(xT, xT, wi_bf)


def _sc_gather(table, idx):
    m, cin = table.shape
    ppw = m // NW
    ch = ppw // CPD
    idx_r = idx.reshape(NW, ch, CI)

    @functools.partial(
        pl.kernel,
        out_type=jax.ShapeDtypeStruct((NW, ch, CI, cin), jnp.float32),
        mesh=plsc.VectorSubcoreMesh(core_axis_name="c", subcore_axis_name="s"),
        compiler_params=pltpu.CompilerParams(use_tc_tiling_on_sc=False),
        scratch_types=[
            pltpu.VMEM((ch, CI), jnp.int32),
            pltpu.VMEM((2, CI, cin), jnp.float32),
            pltpu.SemaphoreType.DMA,
            pltpu.SemaphoreType.DMA,
            pltpu.SemaphoreType.DMA,
            pltpu.SemaphoreType.DMA,
        ],
    )
    def sc_kern(table_hbm, idx_hbm, out_hbm, idx_v, rows_v, gs0, gs1, os0, os1):
        wid = lax.axis_index("s") * 2 + lax.axis_index("c")
        pltpu.sync_copy(idx_hbm.at[wid], idx_v)
        gsems = (gs0, gs1)
        osems = (os0, os1)

        def g_start(c, buf):
            pltpu.async_copy(table_hbm.at[idx_v.at[c]], rows_v.at[buf], gsems[buf])

        def g_wait(c, buf):
            pltpu.make_async_copy(table_hbm.at[idx_v.at[c]], rows_v.at[buf],
                                  gsems[buf]).wait()

        def o_start(c, buf):
            pltpu.async_copy(rows_v.at[buf], out_hbm.at[wid, c], osems[buf])

        def o_wait(c, buf):
            pltpu.make_async_copy(rows_v.at[buf], out_hbm.at[wid, c],
                                  osems[buf]).wait()

        g_start(0, 0)
        g_start(1, 1)

        def loop_body(c2, carry):
            c = 2 * c2
            for buf in (0, 1):
                cc = c + buf
                g_wait(cc, buf)
                o_start(cc, buf)

                @pl.when(cc + 2 < ch)
                def _():
                    o_wait(cc, buf)
                    g_start(cc + 2, buf)

            return carry

        lax.fori_loop(0, ch // 2, loop_body, jnp.int32(0))
        o_wait(ch - 2, 0)
        o_wait(ch - 1, 1)

    return sc_kern(table, idx_r).reshape(m * KNN, cin)


def _edge_body(xj_ref, x_ref, wd_ref, ui_ref, bn_ref, f_ref):
    r = x_ref.shape[1]
    xj = xj_ref[0]
    xi = x_ref[0]
    cin = xi.shape[1]
    g = ui_ref.shape[2]
    xi_b = jnp.broadcast_to(xi[:, None, :], (r, KNN, cin)).reshape(r * KNN, cin)
    e = (xj - xi_b).astype(jnp.bfloat16)
    ed = lax.dot_general(e, wd_ref[...], (((1,), (1,)), ((), ())),
                         preferred_element_type=jnp.float32)
    bias = bn_ref[0][None, None, :]
    gamma = bn_ref[1][None, None, :]
    beta = bn_ref[2][None, None, :]
    y = ed.reshape(r, KNN, g) + ui_ref[0][:, None, :]
    y = y + bias
    y = y / jnp.sqrt(jnp.float32(1.0) + jnp.float32(BN_EPS_K))
    y = y * gamma + beta
    y = jnp.maximum(y, 0.0)
    f_ref[0] = jnp.max(y, axis=1)


def _tc_edge(xj, xT, wd_bf, ui, bn):
    b, n, cin = xT.shape
    g = wd_bf.shape[0]
    r = ROW_TILE
    return pl.pallas_call(
        _edge_body,
        grid=(b, n // r),
        in_specs=[
            pl.BlockSpec((1, r * KNN, cin), lambda i, j: (i, j, 0)),
            pl.BlockSpec((1, r, cin), lambda i, j: (i, j, 0)),
            pl.BlockSpec((g, cin), lambda i, j: (0, 0)),
            pl.BlockSpec((1, r, g), lambda i, j: (i, j, 0)),
            pl.BlockSpec((3, g), lambda i, j: (0, 0)),
        ],
        out_specs=pl.BlockSpec((1, r, g), lambda i, j: (i, j, 0)),
        out_shape=jax.ShapeDtypeStruct((b, n, g), jnp.float32),
    )(xj.reshape(b, n * KNN, cin), xT, wd_bf, ui, bn)


def _edge_block(xT, w, bias, gamma, beta):
    b, n, cin = xT.shape
    g = w.shape[0]
    wi_bf = w[:, :cin].astype(jnp.bfloat16)
    wd_bf = w[:, cin:].astype(jnp.bfloat16)
    bn = jnp.stack([bias, gamma, beta])
    idx, ui = _tc_topk(xT, wi_bf)
    xj = _sc_gather(xT.reshape(b * n, cin), idx.reshape(b * n, KNN))
    return _tc_edge(xj, xT, wd_bf, ui, bn)


def kernel(inputs, W0, b0, gamma0, beta0, W1, b1, gamma1, beta1, W2, b2, gamma2, beta2):
    x0 = jnp.transpose(inputs[..., 0], (0, 2, 1))
    f0 = _edge_block(x0, W0, b0, gamma0, beta0)
    x1 = jnp.concatenate([f0, x0], axis=-1)
    f1 = _edge_block(x1, W1, b1, gamma1, beta1)
    x2 = jnp.concatenate([f1, x1], axis=-1)
    f2 = _edge_block(x2, W2, b2, gamma2, beta2)
    out = jnp.concatenate([f0, f1, f2, x0], axis=-1)
    return jnp.transpose(out, (0, 2, 1))[..., None]
